# baseline (speedup 1.0000x reference)
import functools

import jax
import jax.numpy as jnp
from jax import lax
from jax.experimental import pallas as pl
from jax.experimental.pallas import tpu as pltpu
from jax.experimental.pallas import tpu_sc as plsc

IOU_THR = 0.5
SCORE_THR = 0.05
MAX_OUT = 100
L = 16
KPAD = 128
OP = 112
DELTA = 0.04
BIG = 1 << 30
NEG = float("-inf")

_GDN = lax.GatherDimensionNumbers(offset_dims=(), collapsed_slice_dims=(0,),
                                  start_index_map=(0,))


def _lanes():
    return lax.iota(jnp.int32, L)


def _bc(x):
    return jnp.broadcast_to(x, (L,))


def _shuf(v, idx):
    return lax.gather(v, idx[:, None], _GDN, slice_sizes=(1,),
                      mode=lax.GatherScatterMode.PROMISE_IN_BOUNDS)


def _splatmax(v, lanes):
    for s in (1, 2, 4, 8):
        v = jnp.maximum(v, _shuf(v, lanes ^ s))
    return v


def _splatmin(v, lanes):
    for s in (1, 2, 4, 8):
        v = jnp.minimum(v, _shuf(v, lanes ^ s))
    return v


def _ffs(mask):
    r = plsc.all_reduce_ffs(mask)
    return _bc(r) if r.ndim == 0 else r


@functools.lru_cache(maxsize=None)
def _make_nms(B, C, N, NP):
    NCH4 = NP // (4 * L)
    N4 = N * 4
    BMW = ((N // 32) // L + 2) * L
    mesh = plsc.VectorSubcoreMesh(core_axis_name="core", subcore_axis_name="sub")
    out_type = (
        jax.ShapeDtypeStruct((4, B, OP), jnp.float32),
        jax.ShapeDtypeStruct((B, OP), jnp.float32),
        jax.ShapeDtypeStruct((B, OP), jnp.int32),
    )
    scratch = [
        pltpu.VMEM((NP,), jnp.float32),
        pltpu.VMEM((N4,), jnp.float32),
        pltpu.VMEM((NP + L,), jnp.float32),
        pltpu.VMEM((NP + L,), jnp.int32),
        pltpu.VMEM((KPAD,), jnp.float32),
        pltpu.VMEM((KPAD,), jnp.int32),
        pltpu.VMEM((KPAD,), jnp.float32),
        pltpu.VMEM((KPAD,), jnp.float32),
        pltpu.VMEM((KPAD,), jnp.float32),
        pltpu.VMEM((KPAD,), jnp.float32),
        pltpu.VMEM((KPAD,), jnp.float32),
        pltpu.VMEM_SHARED((C, KPAD), jnp.float32),
        pltpu.VMEM_SHARED((C, KPAD), jnp.int32),
        pltpu.VMEM_SHARED((C, KPAD), jnp.float32),
        pltpu.VMEM_SHARED((C, KPAD), jnp.float32),
        pltpu.VMEM_SHARED((C, KPAD), jnp.float32),
        pltpu.VMEM_SHARED((C, KPAD), jnp.float32),
        pltpu.VMEM((C, KPAD), jnp.float32),
        pltpu.VMEM((C, KPAD), jnp.int32),
        pltpu.VMEM((C, KPAD), jnp.float32),
        pltpu.VMEM((C, KPAD), jnp.float32),
        pltpu.VMEM((C, KPAD), jnp.float32),
        pltpu.VMEM((C, KPAD), jnp.float32),
        pltpu.VMEM((BMW,), jnp.int32),
        pltpu.VMEM((OP,), jnp.float32),
        pltpu.VMEM((OP,), jnp.float32),
        pltpu.VMEM((OP,), jnp.float32),
        pltpu.VMEM((OP,), jnp.float32),
        pltpu.VMEM((OP,), jnp.float32),
        pltpu.VMEM((OP,), jnp.int32),
        pltpu.SemaphoreType.DMA,
    ]

    @functools.partial(pl.kernel, out_type=out_type, mesh=mesh,
                       scratch_types=scratch,
                       compiler_params=pltpu.CompilerParams(
                           needs_layout_passes=False))
    def nms(scores_hbm, boxes_hbm, obh, osh, och,
            s_ref, bx, ext_s, ext_i, ks, kn, ky1, kx1, ky2, kx2, karea,
            sh_ks, sh_kn, sh_y1, sh_x1, sh_y2, sh_x2,
            ks_v, kn_v, vy1, vx1, vy2, vx2,
            bmap, oy1, ox1, oy2, ox2, osc, ocl, bxsem):
        b = lax.axis_index("core")
        c = lax.axis_index("sub")

        @pl.when(c < C)
        def _stage1():
            pltpu.sync_copy(scores_hbm.at[b, c], s_ref)
            bxcopy = pltpu.async_copy(boxes_hbm.at[b], bx, bxsem)

            lanes = _lanes()
            zf = jnp.zeros((L,), jnp.float32)
            zi = jnp.zeros((L,), jnp.int32)
            onesf = jnp.ones((L,), jnp.float32)
            ninf = jnp.full((L,), NEG, jnp.float32)
            bigv = jnp.full((L,), BIG, jnp.int32)
            l15 = jnp.full((L,), 15, jnp.int32)

            for t in range(KPAD // L):
                ks[pl.ds(t * L, L)] = ninf
                kn[pl.ds(t * L, L)] = zi
                ky1[pl.ds(t * L, L)] = zf
                kx1[pl.ds(t * L, L)] = zf
                ky2[pl.ds(t * L, L)] = zf
                kx2[pl.ds(t * L, L)] = zf
                karea[pl.ds(t * L, L)] = onesf

            def m0_body(j, mv):
                m01 = jnp.maximum(s_ref[pl.ds(j * 4 * L, L)],
                                  s_ref[pl.ds(j * 4 * L + L, L)])
                m23 = jnp.maximum(s_ref[pl.ds(j * 4 * L + 2 * L, L)],
                                  s_ref[pl.ds(j * 4 * L + 3 * L, L)])
                return jnp.maximum(mv, jnp.maximum(m01, m23))
            m0 = jnp.max(lax.fori_loop(0, NCH4, m0_body, ninf))
            bxcopy.wait()

            def _consider(pb, mb, kb_, valid_b, nk):
                nb = plsc.load_gather(ext_i, [pb])
                nb4 = nb * 4
                cy1 = plsc.load_gather(bx, [nb4])
                cx1 = plsc.load_gather(bx, [nb4 + 1])
                cy2 = plsc.load_gather(bx, [nb4 + 2])
                cx2 = plsc.load_gather(bx, [nb4 + 3])
                carea = (cy2 - cy1) * (cx2 - cx1)

                def iou_body(v, acc):
                    for h in range(2):
                        d = pl.ds((2 * v + h) * L, L)
                        t1 = jnp.maximum(ky1[d], cy1)
                        t2 = jnp.maximum(kx1[d], cx1)
                        t3 = jnp.minimum(ky2[d], cy2)
                        t4 = jnp.minimum(kx2[d], cx2)
                        inter = (jnp.maximum(t3 - t1, 0.0)
                                 * jnp.maximum(t4 - t2, 0.0))
                        acc = jnp.maximum(acc, inter / (karea[d] + carea
                                                        - inter))
                    return acc
                supb = _splatmax(lax.fori_loop(0, (nk + 1) // 2, iou_body,
                                               zf), lanes) > IOU_THR

                keep_b = valid_b & (~supb)
                wm = (lanes == 0) & keep_b
                plsc.store_scatter(ks, [kb_], mb, mask=wm)
                plsc.store_scatter(kn, [kb_], nb, mask=wm)
                plsc.store_scatter(ky1, [kb_], cy1, mask=wm)
                plsc.store_scatter(kx1, [kb_], cx1, mask=wm)
                plsc.store_scatter(ky2, [kb_], cy2, mask=wm)
                plsc.store_scatter(kx2, [kb_], cx2, mask=wm)
                plsc.store_scatter(karea, [kb_], carea, mask=wm)
                plsc.store_scatter(ext_s, [pb], ninf, mask=lanes == 0)
                return keep_b

            def batch_cond(st):
                hi, kcnt = st
                return (hi > SCORE_THR) & (kcnt < MAX_OUT)

            def batch_body(st):
                hi, kcnt = st
                lo = hi - jnp.float32(DELTA)
                hib = _bc(hi)
                lob = _bc(lo)

                def ext_body(j, base_v):
                    svs, css, wts = [], [], []
                    for u in range(4):
                        sv = s_ref[pl.ds((j * 4 + u) * L, L)]
                        within = (sv > SCORE_THR) & (sv > lob) & (sv <= hib)
                        svs.append(sv)
                        wts.append(within)
                        css.append(jnp.cumsum(jnp.where(within, 1, 0)))
                    for u in range(4):
                        pos = base_v + css[u] - 1
                        plsc.store_scatter(ext_s, [pos], svs[u], mask=wts[u])
                        plsc.store_scatter(ext_i, [pos],
                                           (j * 4 + u) * L + lanes,
                                           mask=wts[u])
                        base_v = base_v + _shuf(css[u], l15)
                    return base_v
                E = jnp.max(lax.fori_loop(0, NCH4, ext_body, zi))
                plsc.store_scatter(ext_s, [_bc(E) + lanes], ninf)
                nv = (E + L - 1) // L

                def sel_cond(st2):
                    consumed, kcnt2 = st2
                    return (consumed < E) & (kcnt2 < MAX_OUT)

                def fast_sel(kcnt_in):
                    def minit_body(v, M):
                        cm = _splatmax(ext_s[pl.ds(v * L, L)], lanes)
                        return jnp.where(lanes == v, cm, M)
                    M0 = lax.fori_loop(0, nv, minit_body, ninf)

                    def cond3(st2):
                        consumed, kcnt_s, kc_v, M = st2
                        return (consumed < E) & (kcnt_s < MAX_OUT)

                    def blk_body(st2):
                        consumed, kcnt_s, kc_v, M = st2
                        nk = (kcnt_s + 4 + L - 1) // L
                        for _ in range(4):
                            mb = _splatmax(M, lanes)
                            valid_b = mb > ninf
                            vstar = _ffs(M == mb)
                            chunk = plsc.load_gather(ext_s,
                                                     [vstar * L + lanes])
                            lstar = _ffs(chunk == mb)
                            pb = vstar * L + lstar
                            keep_b = _consider(pb, mb, kc_v, valid_b, nk)
                            chunk2 = jnp.where(lanes == lstar, ninf, chunk)
                            M = jnp.where(lanes == vstar,
                                          _splatmax(chunk2, lanes), M)
                            kc_v = kc_v + jnp.where(keep_b, 1, 0)
                        return (consumed + 4, jnp.max(kc_v), kc_v, M)

                    _, kc, _, _ = lax.while_loop(
                        cond3, blk_body,
                        (jnp.int32(0), kcnt_in, _bc(kcnt_in), M0))
                    return kc

                def slow_sel(kcnt_in):
                    def sel_body(st2):
                        consumed, kcnt2 = st2

                        def amax_body(v, st3):
                            mv, pv = st3
                            sv = ext_s[pl.ds(v * L, L)]
                            upd = sv > mv
                            pv = jnp.where(upd, v * L + lanes, pv)
                            return jnp.maximum(mv, sv), pv
                        mv, pv = lax.fori_loop(0, nv, amax_body, (ninf, bigv))
                        mb = _splatmax(mv, lanes)
                        p = _splatmin(jnp.where(mv == mb, pv, bigv), lanes)
                        nk2 = (kcnt2 + L - 1) // L
                        keep_b = _consider(p, mb, _bc(kcnt2),
                                           mb > ninf, nk2)
                        keep_s = jnp.max(jnp.where(keep_b, 1, 0))
                        return consumed + 1, kcnt2 + keep_s
                    _, kc = lax.while_loop(sel_cond, sel_body,
                                           (jnp.int32(0), kcnt_in))
                    return kc

                kcnt = lax.cond(nv <= L, fast_sel, slow_sel, kcnt)
                return lo, kcnt

            lax.while_loop(batch_cond, batch_body, (m0, jnp.int32(0)))

            pltpu.sync_copy(ks, sh_ks.at[c])
            pltpu.sync_copy(kn, sh_kn.at[c])
            pltpu.sync_copy(ky1, sh_y1.at[c])
            pltpu.sync_copy(kx1, sh_x1.at[c])
            pltpu.sync_copy(ky2, sh_y2.at[c])
            pltpu.sync_copy(kx2, sh_x2.at[c])

        @pl.when(c == C)
        def _stage2_init():
            zf = jnp.zeros((L,), jnp.float32)
            zi = jnp.zeros((L,), jnp.int32)
            for t in range(BMW // L):
                bmap[pl.ds(t * L, L)] = zi
            for t in range(OP // L):
                oy1[pl.ds(t * L, L)] = zf
                ox1[pl.ds(t * L, L)] = zf
                oy2[pl.ds(t * L, L)] = zf
                ox2[pl.ds(t * L, L)] = zf
                osc[pl.ds(t * L, L)] = zf
                ocl[pl.ds(t * L, L)] = zi

        plsc.subcore_barrier()

        @pl.when(c == C)
        def _stage2():
            pltpu.sync_copy(sh_ks, ks_v)
            pltpu.sync_copy(sh_kn, kn_v)
            pltpu.sync_copy(sh_y1, vy1)
            pltpu.sync_copy(sh_x1, vx1)
            pltpu.sync_copy(sh_y2, vy2)
            pltpu.sync_copy(sh_x2, vx2)

            lanes = _lanes()
            zf = jnp.zeros((L,), jnp.float32)
            zi = jnp.zeros((L,), jnp.int32)
            ninf = jnp.full((L,), NEG, jnp.float32)
            bigv = jnp.full((L,), BIG, jnp.int32)
            onesi = jnp.ones((L,), jnp.int32)
            cmask = lanes < C
            lanesN = lanes * N

            def _emit(cb, pstar, nb, mb, aliveb, ko_v):
                word = plsc.load_gather(bmap, [nb >> 5])
                bit = onesi << (nb & 31)
                dupb = (word & bit) != 0
                plsc.store_scatter(bmap, [nb >> 5], word | bit,
                                   mask=(lanes == 0) & aliveb)
                keepb = aliveb & (~dupb)
                wm = (lanes == 0) & keepb
                plsc.store_scatter(oy1, [ko_v],
                                   plsc.load_gather(vy1, [cb, pstar]), mask=wm)
                plsc.store_scatter(ox1, [ko_v],
                                   plsc.load_gather(vx1, [cb, pstar]), mask=wm)
                plsc.store_scatter(oy2, [ko_v],
                                   plsc.load_gather(vy2, [cb, pstar]), mask=wm)
                plsc.store_scatter(ox2, [ko_v],
                                   plsc.load_gather(vx2, [cb, pstar]), mask=wm)
                plsc.store_scatter(osc, [ko_v], mb, mask=wm)
                plsc.store_scatter(ocl, [ko_v], cb, mask=wm)
                return ko_v + jnp.where(keepb, 1, 0)

            def merge_body(t, st):
                ptrs, ko_v = st
                heads = plsc.load_gather(ks_v, [lanes, ptrs], mask=cmask)
                heads = jnp.where(cmask, heads, ninf)
                nexts = plsc.load_gather(ks_v, [lanes, ptrs + 1], mask=cmask)
                nexts = jnp.where(cmask, nexts, ninf)
                head_n = plsc.load_gather(kn_v, [lanes, ptrs], mask=cmask)
                next_n = plsc.load_gather(kn_v, [lanes, ptrs + 1], mask=cmask)

                mb1 = _splatmax(heads, lanes)
                alive1 = mb1 > jnp.float32(-3e38)
                keyc1 = jnp.where((heads == mb1) & cmask,
                                  lanesN + head_n, bigv)
                sel1 = keyc1 == _splatmin(keyc1, lanes)
                cb1 = jnp.minimum(_ffs(sel1), C - 1)
                nb1 = _shuf(head_n, cb1)
                p1 = jnp.minimum(_shuf(ptrs, cb1), KPAD - 1)

                adv1 = sel1 & alive1 & cmask
                heads2 = jnp.where(adv1, nexts, heads)
                head_n2 = jnp.where(adv1, next_n, head_n)
                ptrs2 = ptrs + jnp.where(adv1, 1, 0)

                mb2 = _splatmax(heads2, lanes)
                alive2 = mb2 > jnp.float32(-3e38)
                keyc2 = jnp.where((heads2 == mb2) & cmask,
                                  lanesN + head_n2, bigv)
                sel2 = keyc2 == _splatmin(keyc2, lanes)
                cb2 = jnp.minimum(_ffs(sel2), C - 1)
                nb2 = _shuf(head_n2, cb2)
                p2 = jnp.minimum(_shuf(ptrs2, cb2), KPAD - 1)
                ptrs3 = ptrs2 + jnp.where(sel2 & alive2 & cmask, 1, 0)

                ko_v = _emit(cb1, p1, nb1, mb1, alive1, ko_v)
                ko_v = _emit(cb2, p2, nb2, mb2, alive2, ko_v)
                return ptrs3, ko_v

            lax.fori_loop(0, MAX_OUT // 2, merge_body, (zi, zi))

            pltpu.sync_copy(oy1, obh.at[0, b])
            pltpu.sync_copy(ox1, obh.at[1, b])
            pltpu.sync_copy(oy2, obh.at[2, b])
            pltpu.sync_copy(ox2, obh.at[3, b])
            pltpu.sync_copy(osc, osh.at[b])
            pltpu.sync_copy(ocl, och.at[b])

    return nms


def kernel(boxes, scores):
    B, N, C = scores.shape
    NP = ((N + 4 * L - 1) // (4 * L)) * (4 * L)
    st = jnp.transpose(scores.astype(jnp.float32), (0, 2, 1))
    st = jnp.pad(st, ((0, 0), (0, 0), (0, NP - N)))
    bflat = boxes.astype(jnp.float32).reshape(B, N * 4)
    ob, osc, ocl = _make_nms(B, C, N, NP)(st, bflat)
    out_boxes = jnp.transpose(ob, (1, 2, 0))[:, :MAX_OUT, :]
    return out_boxes, osc[:, :MAX_OUT], ocl[:, :MAX_OUT]


# speedup vs baseline: 1.0013x; 1.0013x over previous
import functools

import jax
import jax.numpy as jnp
from jax import lax
from jax.experimental import pallas as pl
from jax.experimental.pallas import tpu as pltpu
from jax.experimental.pallas import tpu_sc as plsc

IOU_THR = 0.5
SCORE_THR = 0.05
MAX_OUT = 100
L = 16
KPAD = 128
OP = 112
DELTA = 0.045
BIG = 1 << 30
NEG = float("-inf")

_GDN = lax.GatherDimensionNumbers(offset_dims=(), collapsed_slice_dims=(0,),
                                  start_index_map=(0,))


def _lanes():
    return lax.iota(jnp.int32, L)


def _bc(x):
    return jnp.broadcast_to(x, (L,))


def _shuf(v, idx):
    return lax.gather(v, idx[:, None], _GDN, slice_sizes=(1,),
                      mode=lax.GatherScatterMode.PROMISE_IN_BOUNDS)


def _splatmax(v, lanes):
    for s in (1, 2, 4, 8):
        v = jnp.maximum(v, _shuf(v, lanes ^ s))
    return v


def _splatmin(v, lanes):
    for s in (1, 2, 4, 8):
        v = jnp.minimum(v, _shuf(v, lanes ^ s))
    return v


def _ffs(mask):
    r = plsc.all_reduce_ffs(mask)
    return _bc(r) if r.ndim == 0 else r


@functools.lru_cache(maxsize=None)
def _make_nms(B, C, N, NP):
    NCH4 = NP // (4 * L)
    N4 = N * 4
    BMW = ((N // 32) // L + 2) * L
    mesh = plsc.VectorSubcoreMesh(core_axis_name="core", subcore_axis_name="sub")
    out_type = (
        jax.ShapeDtypeStruct((4, B, OP), jnp.float32),
        jax.ShapeDtypeStruct((B, OP), jnp.float32),
        jax.ShapeDtypeStruct((B, OP), jnp.int32),
    )
    scratch = [
        pltpu.VMEM((NP,), jnp.float32),
        pltpu.VMEM((N4,), jnp.float32),
        pltpu.VMEM((NP + L,), jnp.float32),
        pltpu.VMEM((NP + L,), jnp.int32),
        pltpu.VMEM((KPAD,), jnp.float32),
        pltpu.VMEM((KPAD,), jnp.int32),
        pltpu.VMEM((KPAD,), jnp.float32),
        pltpu.VMEM((KPAD,), jnp.float32),
        pltpu.VMEM((KPAD,), jnp.float32),
        pltpu.VMEM((KPAD,), jnp.float32),
        pltpu.VMEM((KPAD,), jnp.float32),
        pltpu.VMEM_SHARED((C, KPAD), jnp.float32),
        pltpu.VMEM_SHARED((C, KPAD), jnp.int32),
        pltpu.VMEM_SHARED((C, KPAD), jnp.float32),
        pltpu.VMEM_SHARED((C, KPAD), jnp.float32),
        pltpu.VMEM_SHARED((C, KPAD), jnp.float32),
        pltpu.VMEM_SHARED((C, KPAD), jnp.float32),
        pltpu.VMEM((C, KPAD), jnp.float32),
        pltpu.VMEM((C, KPAD), jnp.int32),
        pltpu.VMEM((C, KPAD), jnp.float32),
        pltpu.VMEM((C, KPAD), jnp.float32),
        pltpu.VMEM((C, KPAD), jnp.float32),
        pltpu.VMEM((C, KPAD), jnp.float32),
        pltpu.VMEM((BMW,), jnp.int32),
        pltpu.VMEM((OP,), jnp.float32),
        pltpu.VMEM((OP,), jnp.float32),
        pltpu.VMEM((OP,), jnp.float32),
        pltpu.VMEM((OP,), jnp.float32),
        pltpu.VMEM((OP,), jnp.float32),
        pltpu.VMEM((OP,), jnp.int32),
        pltpu.SemaphoreType.DMA,
    ]

    @functools.partial(pl.kernel, out_type=out_type, mesh=mesh,
                       scratch_types=scratch,
                       compiler_params=pltpu.CompilerParams(
                           needs_layout_passes=False))
    def nms(scores_hbm, boxes_hbm, obh, osh, och,
            s_ref, bx, ext_s, ext_i, ks, kn, ky1, kx1, ky2, kx2, karea,
            sh_ks, sh_kn, sh_y1, sh_x1, sh_y2, sh_x2,
            ks_v, kn_v, vy1, vx1, vy2, vx2,
            bmap, oy1, ox1, oy2, ox2, osc, ocl, bxsem):
        b = lax.axis_index("core")
        c = lax.axis_index("sub")

        @pl.when(c < C)
        def _stage1():
            pltpu.sync_copy(scores_hbm.at[b, c], s_ref)
            bxcopy = pltpu.async_copy(boxes_hbm.at[b], bx, bxsem)

            lanes = _lanes()
            zf = jnp.zeros((L,), jnp.float32)
            zi = jnp.zeros((L,), jnp.int32)
            onesf = jnp.ones((L,), jnp.float32)
            ninf = jnp.full((L,), NEG, jnp.float32)
            bigv = jnp.full((L,), BIG, jnp.int32)
            l15 = jnp.full((L,), 15, jnp.int32)

            for t in range(KPAD // L):
                ks[pl.ds(t * L, L)] = ninf
                kn[pl.ds(t * L, L)] = zi
                ky1[pl.ds(t * L, L)] = zf
                kx1[pl.ds(t * L, L)] = zf
                ky2[pl.ds(t * L, L)] = zf
                kx2[pl.ds(t * L, L)] = zf
                karea[pl.ds(t * L, L)] = onesf

            def m0_body(j, mv):
                m01 = jnp.maximum(s_ref[pl.ds(j * 4 * L, L)],
                                  s_ref[pl.ds(j * 4 * L + L, L)])
                m23 = jnp.maximum(s_ref[pl.ds(j * 4 * L + 2 * L, L)],
                                  s_ref[pl.ds(j * 4 * L + 3 * L, L)])
                return jnp.maximum(mv, jnp.maximum(m01, m23))
            m0 = jnp.max(lax.fori_loop(0, NCH4, m0_body, ninf))
            bxcopy.wait()

            def _consider(pb, mb, kb_, valid_b, nk):
                nb = plsc.load_gather(ext_i, [pb])
                nb4 = nb * 4
                cy1 = plsc.load_gather(bx, [nb4])
                cx1 = plsc.load_gather(bx, [nb4 + 1])
                cy2 = plsc.load_gather(bx, [nb4 + 2])
                cx2 = plsc.load_gather(bx, [nb4 + 3])
                carea = (cy2 - cy1) * (cx2 - cx1)

                def iou_body(v, acc):
                    for h in range(2):
                        d = pl.ds((2 * v + h) * L, L)
                        t1 = jnp.maximum(ky1[d], cy1)
                        t2 = jnp.maximum(kx1[d], cx1)
                        t3 = jnp.minimum(ky2[d], cy2)
                        t4 = jnp.minimum(kx2[d], cx2)
                        inter = (jnp.maximum(t3 - t1, 0.0)
                                 * jnp.maximum(t4 - t2, 0.0))
                        acc = jnp.maximum(acc, inter / (karea[d] + carea
                                                        - inter))
                    return acc
                supb = _splatmax(lax.fori_loop(0, (nk + 1) // 2, iou_body,
                                               zf), lanes) > IOU_THR

                keep_b = valid_b & (~supb)
                wm = (lanes == 0) & keep_b
                plsc.store_scatter(ks, [kb_], mb, mask=wm)
                plsc.store_scatter(kn, [kb_], nb, mask=wm)
                plsc.store_scatter(ky1, [kb_], cy1, mask=wm)
                plsc.store_scatter(kx1, [kb_], cx1, mask=wm)
                plsc.store_scatter(ky2, [kb_], cy2, mask=wm)
                plsc.store_scatter(kx2, [kb_], cx2, mask=wm)
                plsc.store_scatter(karea, [kb_], carea, mask=wm)
                plsc.store_scatter(ext_s, [pb], ninf, mask=lanes == 0)
                return keep_b

            def batch_cond(st):
                hi, kcnt = st
                return (hi > SCORE_THR) & (kcnt < MAX_OUT)

            def batch_body(st):
                hi, kcnt = st
                lo = hi - jnp.float32(DELTA)
                hib = _bc(hi)
                lob = _bc(lo)

                def ext_body(j, base_v):
                    svs, css, wts = [], [], []
                    for u in range(4):
                        sv = s_ref[pl.ds((j * 4 + u) * L, L)]
                        within = (sv > SCORE_THR) & (sv > lob) & (sv <= hib)
                        svs.append(sv)
                        wts.append(within)
                        css.append(jnp.cumsum(jnp.where(within, 1, 0)))
                    for u in range(4):
                        pos = base_v + css[u] - 1
                        plsc.store_scatter(ext_s, [pos], svs[u], mask=wts[u])
                        plsc.store_scatter(ext_i, [pos],
                                           (j * 4 + u) * L + lanes,
                                           mask=wts[u])
                        base_v = base_v + _shuf(css[u], l15)
                    return base_v
                E = jnp.max(lax.fori_loop(0, NCH4, ext_body, zi))
                plsc.store_scatter(ext_s, [_bc(E) + lanes], ninf)
                nv = (E + L - 1) // L

                def sel_cond(st2):
                    consumed, kcnt2 = st2
                    return (consumed < E) & (kcnt2 < MAX_OUT)

                def fast_sel(kcnt_in):
                    def minit_body(v, M):
                        cm = _splatmax(ext_s[pl.ds(v * L, L)], lanes)
                        return jnp.where(lanes == v, cm, M)
                    M0 = lax.fori_loop(0, nv, minit_body, ninf)

                    def cond3(st2):
                        consumed, kcnt_s, kc_v, M = st2
                        return (consumed < E) & (kcnt_s < MAX_OUT)

                    def blk_body(st2):
                        consumed, kcnt_s, kc_v, M = st2
                        nk = (kcnt_s + 4 + L - 1) // L
                        for _ in range(4):
                            mb = _splatmax(M, lanes)
                            valid_b = mb > ninf
                            vstar = _ffs(M == mb)
                            chunk = plsc.load_gather(ext_s,
                                                     [vstar * L + lanes])
                            lstar = _ffs(chunk == mb)
                            pb = vstar * L + lstar
                            keep_b = _consider(pb, mb, kc_v, valid_b, nk)
                            chunk2 = jnp.where(lanes == lstar, ninf, chunk)
                            M = jnp.where(lanes == vstar,
                                          _splatmax(chunk2, lanes), M)
                            kc_v = kc_v + jnp.where(keep_b, 1, 0)
                        return (consumed + 4, jnp.max(kc_v), kc_v, M)

                    _, kc, _, _ = lax.while_loop(
                        cond3, blk_body,
                        (jnp.int32(0), kcnt_in, _bc(kcnt_in), M0))
                    return kc

                def slow_sel(kcnt_in):
                    def sel_body(st2):
                        consumed, kcnt2 = st2

                        def amax_body(v, st3):
                            mv, pv = st3
                            sv = ext_s[pl.ds(v * L, L)]
                            upd = sv > mv
                            pv = jnp.where(upd, v * L + lanes, pv)
                            return jnp.maximum(mv, sv), pv
                        mv, pv = lax.fori_loop(0, nv, amax_body, (ninf, bigv))
                        mb = _splatmax(mv, lanes)
                        p = _splatmin(jnp.where(mv == mb, pv, bigv), lanes)
                        nk2 = (kcnt2 + L - 1) // L
                        keep_b = _consider(p, mb, _bc(kcnt2),
                                           mb > ninf, nk2)
                        keep_s = jnp.max(jnp.where(keep_b, 1, 0))
                        return consumed + 1, kcnt2 + keep_s
                    _, kc = lax.while_loop(sel_cond, sel_body,
                                           (jnp.int32(0), kcnt_in))
                    return kc

                kcnt = lax.cond(nv <= L, fast_sel, slow_sel, kcnt)
                return lo, kcnt

            lax.while_loop(batch_cond, batch_body, (m0, jnp.int32(0)))

            pltpu.sync_copy(ks, sh_ks.at[c])
            pltpu.sync_copy(kn, sh_kn.at[c])
            pltpu.sync_copy(ky1, sh_y1.at[c])
            pltpu.sync_copy(kx1, sh_x1.at[c])
            pltpu.sync_copy(ky2, sh_y2.at[c])
            pltpu.sync_copy(kx2, sh_x2.at[c])

        @pl.when(c == C)
        def _stage2_init():
            zf = jnp.zeros((L,), jnp.float32)
            zi = jnp.zeros((L,), jnp.int32)
            for t in range(BMW // L):
                bmap[pl.ds(t * L, L)] = zi
            for t in range(OP // L):
                oy1[pl.ds(t * L, L)] = zf
                ox1[pl.ds(t * L, L)] = zf
                oy2[pl.ds(t * L, L)] = zf
                ox2[pl.ds(t * L, L)] = zf
                osc[pl.ds(t * L, L)] = zf
                ocl[pl.ds(t * L, L)] = zi

        plsc.subcore_barrier()

        @pl.when(c == C)
        def _stage2():
            pltpu.sync_copy(sh_ks, ks_v)
            pltpu.sync_copy(sh_kn, kn_v)
            pltpu.sync_copy(sh_y1, vy1)
            pltpu.sync_copy(sh_x1, vx1)
            pltpu.sync_copy(sh_y2, vy2)
            pltpu.sync_copy(sh_x2, vx2)

            lanes = _lanes()
            zf = jnp.zeros((L,), jnp.float32)
            zi = jnp.zeros((L,), jnp.int32)
            ninf = jnp.full((L,), NEG, jnp.float32)
            bigv = jnp.full((L,), BIG, jnp.int32)
            onesi = jnp.ones((L,), jnp.int32)
            cmask = lanes < C
            lanesN = lanes * N

            def _emit(cb, pstar, nb, mb, aliveb, ko_v):
                word = plsc.load_gather(bmap, [nb >> 5])
                bit = onesi << (nb & 31)
                dupb = (word & bit) != 0
                plsc.store_scatter(bmap, [nb >> 5], word | bit,
                                   mask=(lanes == 0) & aliveb)
                keepb = aliveb & (~dupb)
                wm = (lanes == 0) & keepb
                plsc.store_scatter(oy1, [ko_v],
                                   plsc.load_gather(vy1, [cb, pstar]), mask=wm)
                plsc.store_scatter(ox1, [ko_v],
                                   plsc.load_gather(vx1, [cb, pstar]), mask=wm)
                plsc.store_scatter(oy2, [ko_v],
                                   plsc.load_gather(vy2, [cb, pstar]), mask=wm)
                plsc.store_scatter(ox2, [ko_v],
                                   plsc.load_gather(vx2, [cb, pstar]), mask=wm)
                plsc.store_scatter(osc, [ko_v], mb, mask=wm)
                plsc.store_scatter(ocl, [ko_v], cb, mask=wm)
                return ko_v + jnp.where(keepb, 1, 0)

            def merge_body(t, st):
                ptrs, ko_v = st
                heads = plsc.load_gather(ks_v, [lanes, ptrs], mask=cmask)
                heads = jnp.where(cmask, heads, ninf)
                nexts = plsc.load_gather(ks_v, [lanes, ptrs + 1], mask=cmask)
                nexts = jnp.where(cmask, nexts, ninf)
                head_n = plsc.load_gather(kn_v, [lanes, ptrs], mask=cmask)
                next_n = plsc.load_gather(kn_v, [lanes, ptrs + 1], mask=cmask)

                mb1 = _splatmax(heads, lanes)
                alive1 = mb1 > jnp.float32(-3e38)
                keyc1 = jnp.where((heads == mb1) & cmask,
                                  lanesN + head_n, bigv)
                sel1 = keyc1 == _splatmin(keyc1, lanes)
                cb1 = jnp.minimum(_ffs(sel1), C - 1)
                nb1 = _shuf(head_n, cb1)
                p1 = jnp.minimum(_shuf(ptrs, cb1), KPAD - 1)

                adv1 = sel1 & alive1 & cmask
                heads2 = jnp.where(adv1, nexts, heads)
                head_n2 = jnp.where(adv1, next_n, head_n)
                ptrs2 = ptrs + jnp.where(adv1, 1, 0)

                mb2 = _splatmax(heads2, lanes)
                alive2 = mb2 > jnp.float32(-3e38)
                keyc2 = jnp.where((heads2 == mb2) & cmask,
                                  lanesN + head_n2, bigv)
                sel2 = keyc2 == _splatmin(keyc2, lanes)
                cb2 = jnp.minimum(_ffs(sel2), C - 1)
                nb2 = _shuf(head_n2, cb2)
                p2 = jnp.minimum(_shuf(ptrs2, cb2), KPAD - 1)
                ptrs3 = ptrs2 + jnp.where(sel2 & alive2 & cmask, 1, 0)

                ko_v = _emit(cb1, p1, nb1, mb1, alive1, ko_v)
                ko_v = _emit(cb2, p2, nb2, mb2, alive2, ko_v)
                return ptrs3, ko_v

            lax.fori_loop(0, MAX_OUT // 2, merge_body, (zi, zi))

            pltpu.sync_copy(oy1, obh.at[0, b])
            pltpu.sync_copy(ox1, obh.at[1, b])
            pltpu.sync_copy(oy2, obh.at[2, b])
            pltpu.sync_copy(ox2, obh.at[3, b])
            pltpu.sync_copy(osc, osh.at[b])
            pltpu.sync_copy(ocl, och.at[b])

    return nms


def kernel(boxes, scores):
    B, N, C = scores.shape
    NP = ((N + 4 * L - 1) // (4 * L)) * (4 * L)
    st = jnp.transpose(scores.astype(jnp.float32), (0, 2, 1))
    st = jnp.pad(st, ((0, 0), (0, 0), (0, NP - N)))
    bflat = boxes.astype(jnp.float32).reshape(B, N * 4)
    ob, osc, ocl = _make_nms(B, C, N, NP)(st, bflat)
    out_boxes = jnp.transpose(ob, (1, 2, 0))[:, :MAX_OUT, :]
    return out_boxes, osc[:, :MAX_OUT], ocl[:, :MAX_OUT]
